# Initial kernel scaffold; baseline (speedup 1.0000x reference)
import jax
import jax.numpy as jnp
from jax.experimental import pallas as pl


def kernel(x, rW1, rb1, rW2, rb2, rW3, rb3, temp, W_pre, b_pre, g_pre, be_pre, W_m1, b_m1, W_m2, b_m2, W_po, b_po, g_po, be_po):
    raise NotImplementedError("write your pallas kernel here")


# speedup vs baseline: 2.7630x; 2.7630x over previous
import functools

import jax
import jax.numpy as jnp
from jax.experimental import pallas as pl
from jax.experimental.pallas import tpu as pltpu

D = 768
HID = 256
TOTAL = 8
MLP_HID = 1536
S = 2048
T = 256
NB = S // T


def _gelu(x):
    return 0.5 * x * (1.0 + jax.lax.erf(x * 0.7071067811865476))


def _ln(x, g, b, eps=1e-5):
    m = jnp.mean(x, axis=-1, keepdims=True)
    v = jnp.mean((x - m) ** 2, axis=-1, keepdims=True)
    return (x - m) / jnp.sqrt(v + eps) * g + b


def _bf(x):
    return x.astype(jnp.bfloat16)


def _dot(a, b):
    return jnp.dot(a, b, preferred_element_type=jnp.float32)


def _body(x_ref, rW1, rb1, rW2, rb2, rW3, rb3, temp,
          Wpre, bpre, gpre, bepre, Wm1, bm1, Wm2, bm2, Wpo, bpo, gpo, bepo,
          out_ref, loss_ref, freq_acc):
    i = pl.program_id(0)
    xb = x_ref[...]

    h = _gelu(_dot(xb, rW1[...]) + rb1[...])
    h = _gelu(_dot(h, rW2[...]) + rb2[...])
    s = _dot(h, rW3[...]) + rb3[...]

    s_max = jnp.max(s, axis=-1, keepdims=True)
    e = jnp.exp(s - s_max)
    p_lb = e / jnp.sum(e, axis=-1, keepdims=True)
    part = jnp.sum(p_lb, axis=0, keepdims=True)

    @pl.when(i == 0)
    def _():
        freq_acc[...] = part

    @pl.when(i > 0)
    def _():
        freq_acc[...] = freq_acc[...] + part

    @pl.when(i == NB - 1)
    def _():
        freq = freq_acc[...] / float(S)
        mu = jnp.mean(freq)
        var = jnp.sum((freq - mu) ** 2) / (TOTAL - 1)
        loss_ref[...] = jnp.reshape(TOTAL * var, (1, 1))

    st = s / temp[...]
    st_max = jnp.max(st, axis=-1, keepdims=True)
    et = jnp.exp(st - st_max)
    pt = et / jnp.sum(et, axis=-1, keepdims=True)

    idx = jax.lax.broadcasted_iota(jnp.int32, (T, TOTAL), 1)
    m1 = jnp.max(pt, axis=-1, keepdims=True)
    i1 = jnp.min(jnp.where(pt == m1, idx, TOTAL), axis=-1, keepdims=True)
    pt2 = jnp.where(idx == i1, -jnp.inf, pt)
    m2 = jnp.max(pt2, axis=-1, keepdims=True)
    i2 = jnp.min(jnp.where(pt2 == m2, idx, TOTAL), axis=-1, keepdims=True)
    w = pt * ((idx == i1) | (idx == i2)).astype(jnp.float32)

    A = []
    for a in range(2):
        z = _dot(_bf(xb), Wpre[a]) + bpre[a]
        z = _ln(z, gpre[a], bepre[a])
        z = _gelu(z) if a == 0 else jnp.maximum(z, 0.0)
        A.append(_bf(z))

    out0 = jnp.zeros((T, D), jnp.float32)
    u1 = jnp.zeros((T, D), jnp.float32)
    sw1 = jnp.zeros((T, 1), jnp.float32)
    for a in range(2):
        for m in range(2):
            h1 = _dot(A[a], Wm1[m]) + bm1[m]
            h1 = _gelu(h1) if m == 0 else jnp.maximum(h1, 0.0)
            xm = _dot(_bf(h1), Wm2[m]) + bm2[m]
            w0 = w[:, a * 4 + m * 2:a * 4 + m * 2 + 1]
            w1 = w[:, a * 4 + m * 2 + 1:a * 4 + m * 2 + 2]
            z0 = _dot(_bf(xm), Wpo[0]) + bpo[0]
            out0 = out0 + w0 * _ln(z0, gpo[...], bepo[...])
            u1 = u1 + w1 * xm
            sw1 = sw1 + w1
    out1 = _dot(_bf(u1), Wpo[1]) + sw1 * bpo[1]
    out_ref[...] = out0 + out1


@functools.partial(jax.jit, static_argnames=())
def kernel(x, rW1, rb1, rW2, rb2, rW3, rb3, temp, W_pre, b_pre, g_pre, be_pre,
           W_m1, b_m1, W_m2, b_m2, W_po, b_po, g_po, be_po):
    xf = x.reshape(S, D)
    full = lambda shape: pl.BlockSpec(shape, lambda i: (0,) * len(shape))
    out, loss = pl.pallas_call(
        _body,
        grid=(NB,),
        in_specs=[
            pl.BlockSpec((T, D), lambda i: (i, 0)),
            full((D, HID)), full((HID,)), full((HID, HID // 2)), full((HID // 2,)),
            full((HID // 2, TOTAL)), full((TOTAL,)), full((1, 1)),
            full((2, D, D)), full((2, D)), full((2, D)), full((2, D)),
            full((2, D, MLP_HID)), full((2, MLP_HID)), full((2, MLP_HID, D)), full((2, D)),
            full((2, D, D)), full((2, D)), full((D,)), full((D,)),
        ],
        out_specs=[
            pl.BlockSpec((T, D), lambda i: (i, 0)),
            pl.BlockSpec((1, 1), lambda i: (0, 0)),
        ],
        out_shape=[
            jax.ShapeDtypeStruct((S, D), jnp.float32),
            jax.ShapeDtypeStruct((1, 1), jnp.float32),
        ],
        scratch_shapes=[pltpu.VMEM((1, TOTAL), jnp.float32)],
        compiler_params=pltpu.CompilerParams(
            dimension_semantics=("arbitrary",),
        ),
    )(
        xf, rW1, rb1, rW2, rb2, rW3, rb3, temp.reshape(1, 1),
        _bf(W_pre), b_pre, g_pre, be_pre,
        _bf(W_m1), b_m1, _bf(W_m2), b_m2,
        _bf(W_po), b_po, g_po, be_po,
    )
    return out.reshape(1, S, D), loss.reshape(())


# speedup vs baseline: 2.7884x; 1.0092x over previous
import functools

import jax
import jax.numpy as jnp
from jax.experimental import pallas as pl
from jax.experimental.pallas import tpu as pltpu

D = 768
HID = 256
TOTAL = 8
MLP_HID = 1536
S = 2048
T = 256
NB = S // T


def _gelu(x):
    return 0.5 * x * (1.0 + jax.lax.erf(x * 0.7071067811865476))


def _ln(x, g, b, eps=1e-5):
    m = jnp.mean(x, axis=-1, keepdims=True)
    v = jnp.mean((x - m) ** 2, axis=-1, keepdims=True)
    return (x - m) / jnp.sqrt(v + eps) * g + b


def _bf(x):
    return x.astype(jnp.bfloat16)


def _dot(a, b):
    return jnp.dot(a, b, preferred_element_type=jnp.float32)


def _body(x_ref, rW1, rb1, rW2, rb2, rW3, rb3, temp,
          Wpre, bpre, gpre, bepre, Wm1, bm1, Wm2, bm2, Wpo, bpo, gpo, bepo,
          out_ref, loss_ref, freq_acc):
    i = pl.program_id(0)
    xb = x_ref[...]

    h = _gelu(_dot(xb, rW1[...]) + rb1[...])
    h = _gelu(_dot(h, rW2[...]) + rb2[...])
    s = _dot(h, rW3[...]) + rb3[...]

    s_max = jnp.max(s, axis=-1, keepdims=True)
    e = jnp.exp(s - s_max)
    p_lb = e / jnp.sum(e, axis=-1, keepdims=True)
    part = jnp.sum(p_lb, axis=0, keepdims=True)

    @pl.when(i == 0)
    def _():
        freq_acc[...] = part

    @pl.when(i > 0)
    def _():
        freq_acc[...] = freq_acc[...] + part

    @pl.when(i == NB - 1)
    def _():
        freq = freq_acc[...] / float(S)
        mu = jnp.mean(freq)
        var = jnp.sum((freq - mu) ** 2) / (TOTAL - 1)
        loss_ref[...] = jnp.reshape(TOTAL * var, (1, 1))

    st = s / temp[...]
    st_max = jnp.max(st, axis=-1, keepdims=True)
    et = jnp.exp(st - st_max)
    pt = et / jnp.sum(et, axis=-1, keepdims=True)

    idx = jax.lax.broadcasted_iota(jnp.int32, (T, TOTAL), 1)
    m1 = jnp.max(pt, axis=-1, keepdims=True)
    i1 = jnp.min(jnp.where(pt == m1, idx, TOTAL), axis=-1, keepdims=True)
    pt2 = jnp.where(idx == i1, -jnp.inf, pt)
    m2 = jnp.max(pt2, axis=-1, keepdims=True)
    i2 = jnp.min(jnp.where(pt2 == m2, idx, TOTAL), axis=-1, keepdims=True)
    w = pt * ((idx == i1) | (idx == i2)).astype(jnp.float32)

    zpre = _dot(_bf(xb), Wpre[...])
    A = []
    for a in range(2):
        z = zpre[:, a * D:(a + 1) * D] + bpre[a]
        z = _ln(z, gpre[a], bepre[a])
        z = _gelu(z) if a == 0 else jnp.maximum(z, 0.0)
        A.append(_bf(z))
    Acat = jnp.concatenate(A, axis=0)

    hcat = _dot(Acat, Wm1[...])
    hm0 = _bf(_gelu(hcat[:, :MLP_HID] + bm1[0]))
    hm1 = _bf(jnp.maximum(hcat[:, MLP_HID:] + bm1[1], 0.0))
    xm0 = _dot(hm0, Wm2[0]) + bm2[0]
    xm1 = _dot(hm1, Wm2[1]) + bm2[1]
    X = jnp.concatenate([xm0, xm1], axis=0)

    z0 = _ln(_dot(_bf(X), Wpo[0]) + bpo[0], gpo[...], bepo[...])
    out0 = jnp.zeros((T, D), jnp.float32)
    u1 = jnp.zeros((T, D), jnp.float32)
    sw1 = jnp.zeros((T, 1), jnp.float32)
    for j in range(4):
        m_i, a_i = j // 2, j % 2
        p0 = a_i * 4 + m_i * 2
        w0 = w[:, p0:p0 + 1]
        w1 = w[:, p0 + 1:p0 + 2]
        out0 = out0 + w0 * z0[j * T:(j + 1) * T]
        u1 = u1 + w1 * X[j * T:(j + 1) * T]
        sw1 = sw1 + w1
    out1 = _dot(_bf(u1), Wpo[1]) + sw1 * bpo[1]
    out_ref[...] = out0 + out1


@functools.partial(jax.jit, static_argnames=())
def kernel(x, rW1, rb1, rW2, rb2, rW3, rb3, temp, W_pre, b_pre, g_pre, be_pre,
           W_m1, b_m1, W_m2, b_m2, W_po, b_po, g_po, be_po):
    xf = x.reshape(S, D)
    full = lambda shape: pl.BlockSpec(shape, lambda i: (0,) * len(shape))
    out, loss = pl.pallas_call(
        _body,
        grid=(NB,),
        in_specs=[
            pl.BlockSpec((T, D), lambda i: (i, 0)),
            full((D, HID)), full((HID,)), full((HID, HID // 2)), full((HID // 2,)),
            full((HID // 2, TOTAL)), full((TOTAL,)), full((1, 1)),
            full((D, 2 * D)), full((2, D)), full((2, D)), full((2, D)),
            full((D, 2 * MLP_HID)), full((2, MLP_HID)), full((2, MLP_HID, D)), full((2, D)),
            full((2, D, D)), full((2, D)), full((D,)), full((D,)),
        ],
        out_specs=[
            pl.BlockSpec((T, D), lambda i: (i, 0)),
            pl.BlockSpec((1, 1), lambda i: (0, 0)),
        ],
        out_shape=[
            jax.ShapeDtypeStruct((S, D), jnp.float32),
            jax.ShapeDtypeStruct((1, 1), jnp.float32),
        ],
        scratch_shapes=[pltpu.VMEM((1, TOTAL), jnp.float32)],
        compiler_params=pltpu.CompilerParams(
            dimension_semantics=("arbitrary",),
        ),
    )(
        xf, rW1, rb1, rW2, rb2, rW3, rb3, temp.reshape(1, 1),
        _bf(jnp.concatenate([W_pre[0], W_pre[1]], axis=1)), b_pre, g_pre, be_pre,
        _bf(jnp.concatenate([W_m1[0], W_m1[1]], axis=1)), b_m1, _bf(W_m2), b_m2,
        _bf(W_po), b_po, g_po, be_po,
    )
    return out.reshape(1, S, D), loss.reshape(())


# speedup vs baseline: 2.9047x; 1.0417x over previous
import functools

import jax
import jax.numpy as jnp
from jax.experimental import pallas as pl
from jax.experimental.pallas import tpu as pltpu

D = 768
HID = 256
TOTAL = 8
MLP_HID = 1536
S = 2048
T = 512
NB = S // T


def _gelu(x):
    return 0.5 * x * (1.0 + jax.lax.erf(x * 0.7071067811865476))


def _ln(x, g, b, eps=1e-5):
    m = jnp.mean(x, axis=-1, keepdims=True)
    v = jnp.mean((x - m) ** 2, axis=-1, keepdims=True)
    return (x - m) / jnp.sqrt(v + eps) * g + b


def _bf(x):
    return x.astype(jnp.bfloat16)


def _dot(a, b):
    return jnp.dot(a, b, preferred_element_type=jnp.float32)


def _body(x_ref, rW1, rb1, rW2, rb2, rW3, rb3, temp,
          Wpre, bpre, gpre, bepre, Wm1, bm1, Wm2, bm2, Wpo, bpo, gpo, bepo,
          out_ref, loss_ref, freq_acc):
    i = pl.program_id(0)
    xb = x_ref[...]

    h = _gelu(_dot(xb, rW1[...]) + rb1[...])
    h = _gelu(_dot(h, rW2[...]) + rb2[...])
    s = _dot(h, rW3[...]) + rb3[...]

    s_max = jnp.max(s, axis=-1, keepdims=True)
    e = jnp.exp(s - s_max)
    p_lb = e / jnp.sum(e, axis=-1, keepdims=True)
    part = jnp.sum(p_lb, axis=0, keepdims=True)

    @pl.when(i == 0)
    def _():
        freq_acc[...] = part

    @pl.when(i > 0)
    def _():
        freq_acc[...] = freq_acc[...] + part

    @pl.when(i == NB - 1)
    def _():
        freq = freq_acc[...] / float(S)
        mu = jnp.mean(freq)
        var = jnp.sum((freq - mu) ** 2) / (TOTAL - 1)
        loss_ref[...] = jnp.reshape(TOTAL * var, (1, 1))

    st = s / temp[...]
    st_max = jnp.max(st, axis=-1, keepdims=True)
    et = jnp.exp(st - st_max)
    pt = et / jnp.sum(et, axis=-1, keepdims=True)

    idx = jax.lax.broadcasted_iota(jnp.int32, (T, TOTAL), 1)
    m1 = jnp.max(pt, axis=-1, keepdims=True)
    i1 = jnp.min(jnp.where(pt == m1, idx, TOTAL), axis=-1, keepdims=True)
    pt2 = jnp.where(idx == i1, -jnp.inf, pt)
    m2 = jnp.max(pt2, axis=-1, keepdims=True)
    i2 = jnp.min(jnp.where(pt2 == m2, idx, TOTAL), axis=-1, keepdims=True)
    w = pt * ((idx == i1) | (idx == i2)).astype(jnp.float32)

    zpre = _dot(_bf(xb), Wpre[...])
    A = []
    for a in range(2):
        z = zpre[:, a * D:(a + 1) * D] + bpre[a]
        z = _ln(z, gpre[a], bepre[a])
        z = _gelu(z) if a == 0 else jnp.maximum(z, 0.0)
        A.append(_bf(z))
    Acat = jnp.concatenate(A, axis=0)

    hcat = _dot(Acat, Wm1[...])
    hm0 = _bf(_gelu(hcat[:, :MLP_HID] + bm1[0]))
    hm1 = _bf(jnp.maximum(hcat[:, MLP_HID:] + bm1[1], 0.0))
    xm0 = _dot(hm0, Wm2[0]) + bm2[0]
    xm1 = _dot(hm1, Wm2[1]) + bm2[1]
    X = jnp.concatenate([xm0, xm1], axis=0)

    z0 = _ln(_dot(_bf(X), Wpo[0]) + bpo[0], gpo[...], bepo[...])
    out0 = jnp.zeros((T, D), jnp.float32)
    u1 = jnp.zeros((T, D), jnp.float32)
    sw1 = jnp.zeros((T, 1), jnp.float32)
    for j in range(4):
        m_i, a_i = j // 2, j % 2
        p0 = a_i * 4 + m_i * 2
        w0 = w[:, p0:p0 + 1]
        w1 = w[:, p0 + 1:p0 + 2]
        out0 = out0 + w0 * z0[j * T:(j + 1) * T]
        u1 = u1 + w1 * X[j * T:(j + 1) * T]
        sw1 = sw1 + w1
    out1 = _dot(_bf(u1), Wpo[1]) + sw1 * bpo[1]
    out_ref[...] = out0 + out1


@functools.partial(jax.jit, static_argnames=())
def kernel(x, rW1, rb1, rW2, rb2, rW3, rb3, temp, W_pre, b_pre, g_pre, be_pre,
           W_m1, b_m1, W_m2, b_m2, W_po, b_po, g_po, be_po):
    xf = x.reshape(S, D)
    full = lambda shape: pl.BlockSpec(shape, lambda i: (0,) * len(shape))
    out, loss = pl.pallas_call(
        _body,
        grid=(NB,),
        in_specs=[
            pl.BlockSpec((T, D), lambda i: (i, 0)),
            full((D, HID)), full((HID,)), full((HID, HID // 2)), full((HID // 2,)),
            full((HID // 2, TOTAL)), full((TOTAL,)), full((1, 1)),
            full((D, 2 * D)), full((2, D)), full((2, D)), full((2, D)),
            full((D, 2 * MLP_HID)), full((2, MLP_HID)), full((2, MLP_HID, D)), full((2, D)),
            full((2, D, D)), full((2, D)), full((D,)), full((D,)),
        ],
        out_specs=[
            pl.BlockSpec((T, D), lambda i: (i, 0)),
            pl.BlockSpec((1, 1), lambda i: (0, 0)),
        ],
        out_shape=[
            jax.ShapeDtypeStruct((S, D), jnp.float32),
            jax.ShapeDtypeStruct((1, 1), jnp.float32),
        ],
        scratch_shapes=[pltpu.VMEM((1, TOTAL), jnp.float32)],
        compiler_params=pltpu.CompilerParams(
            dimension_semantics=("arbitrary",),
        ),
    )(
        xf, rW1, rb1, rW2, rb2, rW3, rb3, temp.reshape(1, 1),
        _bf(jnp.concatenate([W_pre[0], W_pre[1]], axis=1)), b_pre, g_pre, be_pre,
        _bf(jnp.concatenate([W_m1[0], W_m1[1]], axis=1)), b_m1, _bf(W_m2), b_m2,
        _bf(W_po), b_po, g_po, be_po,
    )
    return out.reshape(1, S, D), loss.reshape(())


# speedup vs baseline: 2.9176x; 1.0044x over previous
import functools

import jax
import jax.numpy as jnp
from jax.experimental import pallas as pl
from jax.experimental.pallas import tpu as pltpu

D = 768
HID = 256
TOTAL = 8
MLP_HID = 1536
S = 2048
T = 512
NB = S // T


def _gelu(x):
    return 0.5 * x * (1.0 + jax.lax.erf(x * 0.7071067811865476))


def _ln(x, g, b, eps=1e-5):
    m = jnp.mean(x, axis=-1, keepdims=True)
    v = jnp.mean((x - m) ** 2, axis=-1, keepdims=True)
    return (x - m) / jnp.sqrt(v + eps) * g + b


def _bf(x):
    return x.astype(jnp.bfloat16)


def _dot(a, b):
    return jnp.dot(a, b, preferred_element_type=jnp.float32)


def _body(x_ref, rW1, rb1, rW2, rb2, rW3, rb3, temp,
          Wpre, bpre, gpre, bepre, Wm1, bm1, Wm2, bm2, Wpo, bpo, gpo, bepo,
          out_ref, loss_ref, freq_acc):
    i = pl.program_id(0)
    xb = x_ref[...]

    h = _gelu(_dot(xb, rW1[...]) + rb1[...])
    h = _gelu(_dot(h, rW2[...]) + rb2[...])
    s = _dot(h, rW3[...]) + rb3[...]

    s_max = jnp.max(s, axis=-1, keepdims=True)
    e = jnp.exp(s - s_max)
    p_lb = e / jnp.sum(e, axis=-1, keepdims=True)
    part = jnp.sum(p_lb, axis=0, keepdims=True)

    @pl.when(i == 0)
    def _():
        freq_acc[...] = part

    @pl.when(i > 0)
    def _():
        freq_acc[...] = freq_acc[...] + part

    @pl.when(i == NB - 1)
    def _():
        freq = freq_acc[...] / float(S)
        mu = jnp.mean(freq)
        var = jnp.sum((freq - mu) ** 2) / (TOTAL - 1)
        loss_ref[...] = jnp.reshape(TOTAL * var, (1, 1))

    st = s / temp[...]
    st_max = jnp.max(st, axis=-1, keepdims=True)
    et = jnp.exp(st - st_max)
    pt = et / jnp.sum(et, axis=-1, keepdims=True)

    idx = jax.lax.broadcasted_iota(jnp.int32, (T, TOTAL), 1)
    m1 = jnp.max(pt, axis=-1, keepdims=True)
    i1 = jnp.min(jnp.where(pt == m1, idx, TOTAL), axis=-1, keepdims=True)
    pt2 = jnp.where(idx == i1, -jnp.inf, pt)
    m2 = jnp.max(pt2, axis=-1, keepdims=True)
    i2 = jnp.min(jnp.where(pt2 == m2, idx, TOTAL), axis=-1, keepdims=True)
    w = pt * ((idx == i1) | (idx == i2)).astype(jnp.float32)

    def half(xh, wh):
        th = xh.shape[0]
        zpre = _dot(_bf(xh), Wpre[...])
        A = []
        for a in range(2):
            z = zpre[:, a * D:(a + 1) * D] + bpre[a]
            z = _ln(z, gpre[a], bepre[a])
            z = _gelu(z) if a == 0 else jnp.maximum(z, 0.0)
            A.append(_bf(z))
        Acat = jnp.concatenate(A, axis=0)
        hcat = _dot(Acat, Wm1[...])
        hm0 = _bf(_gelu(hcat[:, :MLP_HID] + bm1[0]))
        hm1 = _bf(jnp.maximum(hcat[:, MLP_HID:] + bm1[1], 0.0))
        xm0 = _dot(hm0, Wm2[0]) + bm2[0]
        xm1 = _dot(hm1, Wm2[1]) + bm2[1]
        X = jnp.concatenate([xm0, xm1], axis=0)
        z0 = _ln(_dot(_bf(X), Wpo[0]) + bpo[0], gpo[...], bepo[...])
        out0 = jnp.zeros((th, D), jnp.float32)
        u1 = jnp.zeros((th, D), jnp.float32)
        sw1 = jnp.zeros((th, 1), jnp.float32)
        for j in range(4):
            m_i, a_i = j // 2, j % 2
            p0 = a_i * 4 + m_i * 2
            w0 = wh[:, p0:p0 + 1]
            w1 = wh[:, p0 + 1:p0 + 2]
            out0 = out0 + w0 * z0[j * th:(j + 1) * th]
            u1 = u1 + w1 * X[j * th:(j + 1) * th]
            sw1 = sw1 + w1
        return out0 + _dot(_bf(u1), Wpo[1]) + sw1 * bpo[1]

    H = T // 2
    o1 = half(xb[:H], w[:H])
    o2 = half(xb[H:], w[H:])
    out_ref[...] = jnp.concatenate([o1, o2], axis=0)


@functools.partial(jax.jit, static_argnames=())
def kernel(x, rW1, rb1, rW2, rb2, rW3, rb3, temp, W_pre, b_pre, g_pre, be_pre,
           W_m1, b_m1, W_m2, b_m2, W_po, b_po, g_po, be_po):
    xf = x.reshape(S, D)
    full = lambda shape: pl.BlockSpec(shape, lambda i: (0,) * len(shape))
    out, loss = pl.pallas_call(
        _body,
        grid=(NB,),
        in_specs=[
            pl.BlockSpec((T, D), lambda i: (i, 0)),
            full((D, HID)), full((HID,)), full((HID, HID // 2)), full((HID // 2,)),
            full((HID // 2, TOTAL)), full((TOTAL,)), full((1, 1)),
            full((D, 2 * D)), full((2, D)), full((2, D)), full((2, D)),
            full((D, 2 * MLP_HID)), full((2, MLP_HID)), full((2, MLP_HID, D)), full((2, D)),
            full((2, D, D)), full((2, D)), full((D,)), full((D,)),
        ],
        out_specs=[
            pl.BlockSpec((T, D), lambda i: (i, 0)),
            pl.BlockSpec((1, 1), lambda i: (0, 0)),
        ],
        out_shape=[
            jax.ShapeDtypeStruct((S, D), jnp.float32),
            jax.ShapeDtypeStruct((1, 1), jnp.float32),
        ],
        scratch_shapes=[pltpu.VMEM((1, TOTAL), jnp.float32)],
        compiler_params=pltpu.CompilerParams(
            dimension_semantics=("arbitrary",),
        ),
    )(
        xf, rW1, rb1, rW2, rb2, rW3, rb3, temp.reshape(1, 1),
        _bf(jnp.concatenate([W_pre[0], W_pre[1]], axis=1)), b_pre, g_pre, be_pre,
        _bf(jnp.concatenate([W_m1[0], W_m1[1]], axis=1)), b_m1, _bf(W_m2), b_m2,
        _bf(W_po), b_po, g_po, be_po,
    )
    return out.reshape(1, S, D), loss.reshape(())


# speedup vs baseline: 3.0201x; 1.0351x over previous
import functools

import jax
import jax.numpy as jnp
from jax.experimental import pallas as pl
from jax.experimental.pallas import tpu as pltpu

D = 768
HID = 256
TOTAL = 8
MLP_HID = 1536
S = 2048
T = 256
NB = S // T


def _gelu(x):
    return 0.5 * x * (1.0 + jax.lax.erf(x * 0.7071067811865476))


def _ln(x, g, b, eps=1e-5):
    m = jnp.mean(x, axis=-1, keepdims=True)
    v = jnp.mean((x - m) ** 2, axis=-1, keepdims=True)
    return (x - m) / jnp.sqrt(v + eps) * g + b


def _bf(x):
    return x.astype(jnp.bfloat16)


def _dot(a, b):
    return jnp.dot(a, b, preferred_element_type=jnp.float32)


def _body(x_ref, rW1, rb1, rW2, rb2, rW3, rb3, temp,
          Wpre_h, bpre, gpre, bepre, Wm1_h, bm1, Wm2_h, bm2, Wpo_h, bpo,
          gpo, bepo,
          out_ref, loss_ref,
          freq_acc, Wpre_b, Wm1_b, Wm2_b, Wpo_b, stg0, stg1, sem):
    i = pl.program_id(0)

    @pl.when(i == 0)
    def _():
        stg = (stg0, stg1)
        chunks = [
            (Wpre_h.at[0], D, D), (Wpre_h.at[1], D, D),
            (Wm1_h.at[0], D, MLP_HID), (Wm1_h.at[1], D, MLP_HID),
            (Wm2_h.at[0], MLP_HID, D), (Wm2_h.at[1], MLP_HID, D),
            (Wpo_h.at[0], D, D), (Wpo_h.at[1], D, D),
        ]

        def store_chunk(k, v):
            if k == 0:
                Wpre_b[:, :D] = v
            elif k == 1:
                Wpre_b[:, D:] = v
            elif k == 2:
                Wm1_b[:, :MLP_HID] = v
            elif k == 3:
                Wm1_b[:, MLP_HID:] = v
            elif k == 4:
                Wm2_b[0] = v
            elif k == 5:
                Wm2_b[1] = v
            elif k == 6:
                Wpo_b[0] = v
            else:
                Wpo_b[1] = v

        copies = []
        for k, (src, r, c) in enumerate(chunks):
            copies.append(pltpu.make_async_copy(
                src, stg[k % 2].at[:r, :c], sem.at[k % 2]))
        copies[0].start()
        for k, (src, r, c) in enumerate(chunks):
            if k + 1 < len(chunks):
                copies[k + 1].start()
            copies[k].wait()
            store_chunk(k, _bf(stg[k % 2][:r, :c]))

    xb = x_ref[...]

    h = _gelu(_dot(xb, rW1[...]) + rb1[...])
    h = _gelu(_dot(h, rW2[...]) + rb2[...])
    s = _dot(h, rW3[...]) + rb3[...]

    s_max = jnp.max(s, axis=-1, keepdims=True)
    e = jnp.exp(s - s_max)
    p_lb = e / jnp.sum(e, axis=-1, keepdims=True)
    part = jnp.sum(p_lb, axis=0, keepdims=True)

    @pl.when(i == 0)
    def _():
        freq_acc[...] = part

    @pl.when(i > 0)
    def _():
        freq_acc[...] = freq_acc[...] + part

    @pl.when(i == NB - 1)
    def _():
        freq = freq_acc[...] / float(S)
        mu = jnp.mean(freq)
        var = jnp.sum((freq - mu) ** 2) / (TOTAL - 1)
        loss_ref[...] = jnp.reshape(TOTAL * var, (1, 1))

    st = s / temp[...]
    st_max = jnp.max(st, axis=-1, keepdims=True)
    et = jnp.exp(st - st_max)
    pt = et / jnp.sum(et, axis=-1, keepdims=True)

    idx = jax.lax.broadcasted_iota(jnp.int32, (T, TOTAL), 1)
    m1 = jnp.max(pt, axis=-1, keepdims=True)
    i1 = jnp.min(jnp.where(pt == m1, idx, TOTAL), axis=-1, keepdims=True)
    pt2 = jnp.where(idx == i1, -jnp.inf, pt)
    m2 = jnp.max(pt2, axis=-1, keepdims=True)
    i2 = jnp.min(jnp.where(pt2 == m2, idx, TOTAL), axis=-1, keepdims=True)
    w = pt * ((idx == i1) | (idx == i2)).astype(jnp.float32)

    def half(xh, wh):
        th = xh.shape[0]
        zpre = _dot(_bf(xh), Wpre_b[...])
        A = []
        for a in range(2):
            z = zpre[:, a * D:(a + 1) * D] + bpre[a]
            z = _ln(z, gpre[a], bepre[a])
            z = _gelu(z) if a == 0 else jnp.maximum(z, 0.0)
            A.append(_bf(z))
        Acat = jnp.concatenate(A, axis=0)
        hcat = _dot(Acat, Wm1_b[...])
        hm0 = _bf(_gelu(hcat[:, :MLP_HID] + bm1[0]))
        hm1 = _bf(jnp.maximum(hcat[:, MLP_HID:] + bm1[1], 0.0))
        xm0 = _dot(hm0, Wm2_b[0]) + bm2[0]
        xm1 = _dot(hm1, Wm2_b[1]) + bm2[1]
        X = jnp.concatenate([xm0, xm1], axis=0)
        z0 = _ln(_dot(_bf(X), Wpo_b[0]) + bpo[0], gpo[...], bepo[...])
        out0 = jnp.zeros((th, D), jnp.float32)
        u1 = jnp.zeros((th, D), jnp.float32)
        sw1 = jnp.zeros((th, 1), jnp.float32)
        for j in range(4):
            m_i, a_i = j // 2, j % 2
            p0 = a_i * 4 + m_i * 2
            w0 = wh[:, p0:p0 + 1]
            w1 = wh[:, p0 + 1:p0 + 2]
            out0 = out0 + w0 * z0[j * th:(j + 1) * th]
            u1 = u1 + w1 * X[j * th:(j + 1) * th]
            sw1 = sw1 + w1
        return out0 + _dot(_bf(u1), Wpo_b[1]) + sw1 * bpo[1]

    H = T // 2
    o1 = half(xb[:H], w[:H])
    o2 = half(xb[H:], w[H:])
    out_ref[...] = jnp.concatenate([o1, o2], axis=0)


@functools.partial(jax.jit, static_argnames=())
def kernel(x, rW1, rb1, rW2, rb2, rW3, rb3, temp, W_pre, b_pre, g_pre, be_pre,
           W_m1, b_m1, W_m2, b_m2, W_po, b_po, g_po, be_po):
    xf = x.reshape(S, D)
    full = lambda shape: pl.BlockSpec(shape, lambda i: (0,) * len(shape))
    hbm = pl.BlockSpec(memory_space=pl.ANY)
    out, loss = pl.pallas_call(
        _body,
        grid=(NB,),
        in_specs=[
            pl.BlockSpec((T, D), lambda i: (i, 0)),
            full((D, HID)), full((HID,)), full((HID, HID // 2)), full((HID // 2,)),
            full((HID // 2, TOTAL)), full((TOTAL,)), full((1, 1)),
            hbm, full((2, D)), full((2, D)), full((2, D)),
            hbm, full((2, MLP_HID)), hbm, full((2, D)),
            hbm, full((2, D)), full((D,)), full((D,)),
        ],
        out_specs=[
            pl.BlockSpec((T, D), lambda i: (i, 0)),
            pl.BlockSpec((1, 1), lambda i: (0, 0)),
        ],
        out_shape=[
            jax.ShapeDtypeStruct((S, D), jnp.float32),
            jax.ShapeDtypeStruct((1, 1), jnp.float32),
        ],
        scratch_shapes=[
            pltpu.VMEM((1, TOTAL), jnp.float32),
            pltpu.VMEM((D, 2 * D), jnp.bfloat16),
            pltpu.VMEM((D, 2 * MLP_HID), jnp.bfloat16),
            pltpu.VMEM((2, MLP_HID, D), jnp.bfloat16),
            pltpu.VMEM((2, D, D), jnp.bfloat16),
            pltpu.VMEM((MLP_HID, MLP_HID), jnp.float32),
            pltpu.VMEM((MLP_HID, MLP_HID), jnp.float32),
            pltpu.SemaphoreType.DMA((2,)),
        ],
        compiler_params=pltpu.CompilerParams(
            dimension_semantics=("arbitrary",),
        ),
    )(
        xf, rW1, rb1, rW2, rb2, rW3, rb3, temp.reshape(1, 1),
        W_pre, b_pre, g_pre, be_pre,
        W_m1, b_m1, W_m2, b_m2,
        W_po, b_po, g_po, be_po,
    )
    return out.reshape(1, S, D), loss.reshape(())
